# baseline (speedup 1.0000x reference)
import jax
import jax.numpy as jnp
from jax import lax
from jax.experimental import pallas as pl
from jax.experimental.pallas import tpu as pltpu
from jax.experimental.pallas import tpu_sc as plsc

B = 16384
C = 1000
NC = 2
NS = 16
NW = NC * NS
BPW = B // NW
R = 32
NCH = BPW // R
NBUF = 3

_IN_BOUNDS = "promise_in_bounds"
_LN2 = 0.6931471805599453
_SQRT2 = 1.4142135623730951


def _rot(x, lanes, sh):
    return x.at[(lanes + sh) & 15].get(mode=_IN_BOUNDS)


def _allsum(x, lanes):
    for sh in (8, 4, 2, 1):
        x = x + _rot(x, lanes, sh)
    return x


def _ln(r):
    bits = lax.bitcast_convert_type(r, jnp.int32)
    e = ((bits >> 23) & 0xFF) - 127
    m = lax.bitcast_convert_type((bits & 0x007FFFFF) | 0x3F800000,
                                 jnp.float32)
    big = m > _SQRT2
    m = jnp.where(big, m * 0.5, m)
    e = jnp.where(big, e + 1, e)
    f = m - 1.0
    s = f / (2.0 + f)
    s2 = s * s
    p = 2.0 * s * (1.0 + s2 * (1.0 / 3.0 + s2 * (0.2 + s2 * (1.0 / 7.0))))
    return e.astype(jnp.float32) * _LN2 + p


def _sc_body(x_hbm, tgt_hbm, out_hbm,
             tgt_v, b0, b1, b2, out_v, psum_v, s0, s1, s2):
    bufs = [b0, b1, b2]
    sems = [s0, s1, s2]
    wid = lax.axis_index("s") * NC + lax.axis_index("c")
    base = wid * BPW
    lanes = lax.iota(jnp.int32, 16)
    zeros16 = jnp.zeros((16,), jnp.int32)

    pltpu.sync_copy(tgt_hbm.at[pl.ds(base, BPW)], tgt_v)

    def start(ch):
        p = ch % NBUF
        return pltpu.async_copy(
            x_hbm.at[pl.ds(base + ch * R, R)],
            bufs[p], sems[p])

    handles = {}
    for ch in range(NBUF - 1):
        handles[ch] = start(ch)

    for ch in range(NCH):
        if ch + NBUF - 1 < NCH:
            handles[ch + NBUF - 1] = start(ch + NBUF - 1)
        handles[ch].wait()
        buf = bufs[ch % NBUF]
        for gg in range(R // 16):
            g = ch * (R // 16) + gg
            sl = pl.ds(g * 16, 16)
            rloc = gg * 16 + lanes
            t16 = tgt_v[sl]
            xt = plsc.load_gather(buf, [rloc, t16])
            x0 = plsc.load_gather(buf, [rloc, zeros16])
            out16 = 1.0 + jnp.exp(x0 - xt)
            out_v[sl] = out16
            zmask = t16 == 0
            nz = _allsum(zmask.astype(jnp.int32), lanes)[0]

            @pl.when(nz > 0)
            def _():
                def colbody(c, acc):
                    col = plsc.load_gather(buf, [rloc,
                                                 jnp.broadcast_to(c, (16,))])
                    return acc + jnp.exp(col - x0)
                acc = lax.fori_loop(0, C, colbody,
                                    jnp.zeros((16,), jnp.float32))
                out_v[sl] = jnp.where(zmask, acc, out16)

    acc_ln = jnp.zeros((16,), jnp.float32)
    for g in range(BPW // 16):
        acc_ln = acc_ln + _ln(out_v[pl.ds(g * 16, 16)])
    psum_v[...] = acc_ln
    pltpu.sync_copy(psum_v, out_hbm.at[wid])


@jax.jit
def _sc_loss_partials(x2d, tgt):
    mesh = plsc.VectorSubcoreMesh(core_axis_name="c", subcore_axis_name="s",
                                  num_cores=NC, num_subcores=NS)
    return pl.kernel(
        _sc_body,
        out_type=jax.ShapeDtypeStruct((NW, 16), jnp.float32),
        mesh=mesh,
        scratch_types=[
            pltpu.VMEM((BPW,), jnp.int32),
            pltpu.VMEM((R, C), jnp.float32),
            pltpu.VMEM((R, C), jnp.float32),
            pltpu.VMEM((R, C), jnp.float32),
            pltpu.VMEM((BPW,), jnp.float32),
            pltpu.VMEM((16,), jnp.float32),
            pltpu.SemaphoreType.DMA,
            pltpu.SemaphoreType.DMA,
            pltpu.SemaphoreType.DMA,
        ],
        compiler_params=pltpu.CompilerParams(
            needs_layout_passes=False,
            use_tc_tiling_on_sc=False,
        ),
    )(x2d, tgt)


def kernel(inputs, targets):
    tgt = targets.astype(jnp.int32)
    partials = _sc_loss_partials(inputs, tgt)
    return jnp.sum(partials) * (1.0 / B)


# speedup vs baseline: 1.1873x; 1.1873x over previous
import jax
import jax.numpy as jnp
from jax import lax
from jax.experimental import pallas as pl
from jax.experimental.pallas import tpu as pltpu
from jax.experimental.pallas import tpu_sc as plsc

B = 16384
C = 1000
NC = 2
NS = 16
NW = NC * NS
BPW = B // NW
NG = BPW // 16
ROWPAD = (C + 15) // 16 * 16

_IN_BOUNDS = "promise_in_bounds"
_LN2 = 0.6931471805599453
_SQRT2 = 1.4142135623730951


def _rot(x, lanes, sh):
    return x.at[(lanes + sh) & 15].get(mode=_IN_BOUNDS)


def _allsum(x, lanes):
    for sh in (8, 4, 2, 1):
        x = x + _rot(x, lanes, sh)
    return x


def _prefix_sum(x, lanes):
    zero = jnp.zeros_like(x)
    for sh in (1, 2, 4, 8):
        y = _rot(x, lanes, -sh)
        x = x + jnp.where(lanes >= sh, y, zero)
    return x


def _ln(r):
    bits = lax.bitcast_convert_type(r, jnp.int32)
    e = ((bits >> 23) & 0xFF) - 127
    m = lax.bitcast_convert_type((bits & 0x007FFFFF) | 0x3F800000,
                                 jnp.float32)
    big = m > _SQRT2
    m = jnp.where(big, m * 0.5, m)
    e = jnp.where(big, e + 1, e)
    f = m - 1.0
    s = f / (2.0 + f)
    s2 = s * s
    p = 2.0 * s * (1.0 + s2 * (1.0 / 3.0 + s2 * (0.2 + s2 * (1.0 / 7.0))))
    return e.astype(jnp.float32) * _LN2 + p


def _sc_body(flat_hbm, tgt_hbm, out_hbm,
             tgt_v, idx_t, idx_0, xt_v, x0_v, zrows_v, out_v, rowbuf_v,
             psum_v, sem):
    wid = lax.axis_index("s") * NC + lax.axis_index("c")
    base = wid * BPW
    lanes = lax.iota(jnp.int32, 16)

    pltpu.sync_copy(tgt_hbm.at[pl.ds(base, BPW)], tgt_v)

    cursor = jnp.zeros((16,), jnp.int32)
    for g in range(NG):
        t16 = tgt_v[pl.ds(g * 16, 16)]
        rows_loc = g * 16 + lanes
        row_base = (base + rows_loc) * C
        idx_t[pl.ds(g * 16, 16)] = row_base + t16
        idx_0[pl.ds(g * 16, 16)] = row_base
        zmask = t16 == 0
        zint = zmask.astype(jnp.int32)
        pos = _prefix_sum(zint, lanes)
        zidx = jnp.where(zmask, cursor + pos - 1, 0)
        plsc.store_scatter(zrows_v, [zidx], rows_loc, mask=zmask)
        cursor = cursor + _allsum(zint, lanes)
    n0 = cursor[0]

    copies = []
    for k in range(BPW // 128):
        sl = pl.ds(k * 128, 128)
        copies.append(pltpu.async_copy(flat_hbm.at[idx_t.at[sl]],
                                       xt_v.at[sl], sem))
        copies.append(pltpu.async_copy(flat_hbm.at[idx_0.at[sl]],
                                       x0_v.at[sl], sem))
    for cp in copies:
        cp.wait()

    for g in range(NG):
        sl = pl.ds(g * 16, 16)
        out_v[sl] = 1.0 + jnp.exp(x0_v[sl] - xt_v[sl])

    def zrow_body(j, carry):
        j16 = jnp.broadcast_to(j, (16,)).astype(jnp.int32)
        row_loc = plsc.load_gather(zrows_v, [j16])[0]
        off = (base + row_loc) * C
        pltpu.sync_copy(flat_hbm.at[pl.ds(off, C)], rowbuf_v.at[pl.ds(0, C)])
        x0s = plsc.load_gather(rowbuf_v, [jnp.zeros((16,), jnp.int32)])
        acc = jnp.zeros((16,), jnp.float32)
        for k in range(ROWPAD // 16):
            v = rowbuf_v[pl.ds(k * 16, 16)]
            if (k + 1) * 16 > C:
                v = jnp.where(lanes < C - k * 16, v, -1e30)
            acc = acc + jnp.exp(v - x0s)
        tot = _allsum(acc, lanes)
        plsc.store_scatter(out_v, [jnp.broadcast_to(row_loc, (16,))], tot,
                           mask=lanes == 0)
        return carry

    lax.fori_loop(0, n0, zrow_body, 0)

    acc_ln = jnp.zeros((16,), jnp.float32)
    for g in range(NG):
        acc_ln = acc_ln + _ln(out_v[pl.ds(g * 16, 16)])
    psum_v[...] = acc_ln
    pltpu.sync_copy(psum_v, out_hbm.at[wid])


@jax.jit
def _sc_loss_partials(flat, tgt):
    mesh = plsc.VectorSubcoreMesh(core_axis_name="c", subcore_axis_name="s",
                                  num_cores=NC, num_subcores=NS)
    return pl.kernel(
        _sc_body,
        out_type=jax.ShapeDtypeStruct((NW, 16), jnp.float32),
        mesh=mesh,
        scratch_types=[
            pltpu.VMEM((BPW,), jnp.int32),
            pltpu.VMEM((BPW,), jnp.int32),
            pltpu.VMEM((BPW,), jnp.int32),
            pltpu.VMEM((BPW,), jnp.float32),
            pltpu.VMEM((BPW,), jnp.float32),
            pltpu.VMEM((BPW,), jnp.int32),
            pltpu.VMEM((BPW,), jnp.float32),
            pltpu.VMEM((ROWPAD,), jnp.float32),
            pltpu.VMEM((16,), jnp.float32),
            pltpu.SemaphoreType.DMA,
        ],
        compiler_params=pltpu.CompilerParams(needs_layout_passes=False),
    )(flat, tgt)


def kernel(inputs, targets):
    flat = inputs.reshape(B * C)
    tgt = targets.astype(jnp.int32)
    partials = _sc_loss_partials(flat, tgt)
    return jnp.sum(partials) * (1.0 / B)


# speedup vs baseline: 1.1949x; 1.0064x over previous
import jax
import jax.numpy as jnp
from jax import lax
from jax.experimental import pallas as pl
from jax.experimental.pallas import tpu as pltpu
from jax.experimental.pallas import tpu_sc as plsc

B = 16384
C = 1000
NC = 2
NS = 16
NW = NC * NS
BPW = B // NW
NG = BPW // 16
ROWPAD = (C + 15) // 16 * 16

_IN_BOUNDS = "promise_in_bounds"
_LN2 = 0.6931471805599453
_SQRT2 = 1.4142135623730951


def _rot(x, lanes, sh):
    return x.at[(lanes + sh) & 15].get(mode=_IN_BOUNDS)


def _allsum(x, lanes):
    for sh in (8, 4, 2, 1):
        x = x + _rot(x, lanes, sh)
    return x


def _prefix_sum(x, lanes):
    zero = jnp.zeros_like(x)
    for sh in (1, 2, 4, 8):
        y = _rot(x, lanes, -sh)
        x = x + jnp.where(lanes >= sh, y, zero)
    return x


def _ln(r):
    bits = lax.bitcast_convert_type(r, jnp.int32)
    e = ((bits >> 23) & 0xFF) - 127
    m = lax.bitcast_convert_type((bits & 0x007FFFFF) | 0x3F800000,
                                 jnp.float32)
    big = m > _SQRT2
    m = jnp.where(big, m * 0.5, m)
    e = jnp.where(big, e + 1, e)
    f = m - 1.0
    s = f / (2.0 + f)
    s2 = s * s
    p = 2.0 * s * (1.0 + s2 * (1.0 / 3.0 + s2 * (0.2 + s2 * (1.0 / 7.0))))
    return e.astype(jnp.float32) * _LN2 + p


def _sc_body(flat_hbm, tgt_hbm, out_hbm,
             tgt_v, idx_t, idx_0, xt_v, x0_v, zrows_v, out_v, rowbuf_v,
             psum_v, sem):
    wid = lax.axis_index("s") * NC + lax.axis_index("c")
    base = wid * BPW
    lanes = lax.iota(jnp.int32, 16)

    pltpu.sync_copy(tgt_hbm.at[pl.ds(base, BPW)], tgt_v)

    cursor = jnp.zeros((16,), jnp.int32)
    for g in range(NG):
        t16 = tgt_v[pl.ds(g * 16, 16)]
        rows_loc = g * 16 + lanes
        row_base = (base + rows_loc) * C
        idx_t[pl.ds(g * 16, 16)] = row_base + t16
        idx_0[pl.ds(g * 16, 16)] = row_base
        zmask = t16 == 0
        zint = zmask.astype(jnp.int32)
        pos = _prefix_sum(zint, lanes)
        zidx = jnp.where(zmask, cursor + pos - 1, 0)
        plsc.store_scatter(zrows_v, [zidx], rows_loc, mask=zmask)
        cursor = cursor + _allsum(zint, lanes)
    n0 = cursor[0]

    copies = []
    for k in range(BPW // 128):
        sl = pl.ds(k * 128, 128)
        copies.append(pltpu.async_copy(flat_hbm.at[idx_t.at[sl]],
                                       xt_v.at[sl], sem))
        copies.append(pltpu.async_copy(flat_hbm.at[idx_0.at[sl]],
                                       x0_v.at[sl], sem))
    for cp in copies:
        cp.wait()

    for g in range(NG):
        sl = pl.ds(g * 16, 16)
        out_v[sl] = 1.0 + jnp.exp(x0_v[sl] - xt_v[sl])

    def zrow_body(j, carry):
        j16 = jnp.broadcast_to(j, (16,)).astype(jnp.int32)
        row_loc = plsc.load_gather(zrows_v, [j16])[0]
        off = (base + row_loc) * C
        pltpu.sync_copy(flat_hbm.at[pl.ds(off, C)], rowbuf_v.at[pl.ds(0, C)])
        x0s = plsc.load_gather(rowbuf_v, [jnp.zeros((16,), jnp.int32)])
        acc = jnp.zeros((16,), jnp.float32)
        for k in range(ROWPAD // 16):
            v = rowbuf_v[pl.ds(k * 16, 16)]
            if (k + 1) * 16 > C:
                v = jnp.where(lanes < C - k * 16, v, -1e30)
            acc = acc + jnp.exp(v - x0s)
        tot = _allsum(acc, lanes)
        plsc.store_scatter(out_v, [jnp.broadcast_to(row_loc, (16,))], tot,
                           mask=lanes == 0)
        return carry

    lax.fori_loop(0, n0, zrow_body, 0)

    acc_ln = jnp.zeros((16,), jnp.float32)
    for g in range(NG):
        acc_ln = acc_ln + _ln(out_v[pl.ds(g * 16, 16)])
    psum_v[...] = acc_ln
    pltpu.sync_copy(psum_v, out_hbm.at[wid])


@jax.jit
def _sc_loss_partials(flat, tgt):
    mesh = plsc.VectorSubcoreMesh(core_axis_name="c", subcore_axis_name="s",
                                  num_cores=NC, num_subcores=NS)
    return pl.kernel(
        _sc_body,
        out_type=jax.ShapeDtypeStruct((NW, 16), jnp.float32),
        mesh=mesh,
        scratch_types=[
            pltpu.VMEM((BPW,), jnp.int32),
            pltpu.VMEM((BPW,), jnp.int32),
            pltpu.VMEM((BPW,), jnp.int32),
            pltpu.VMEM((BPW,), jnp.float32),
            pltpu.VMEM((BPW,), jnp.float32),
            pltpu.VMEM((BPW,), jnp.int32),
            pltpu.VMEM((BPW,), jnp.float32),
            pltpu.VMEM((ROWPAD,), jnp.float32),
            pltpu.VMEM((16,), jnp.float32),
            pltpu.SemaphoreType.DMA,
        ],
        compiler_params=pltpu.CompilerParams(needs_layout_passes=False),
    )(flat, tgt)


def _mean_body(p_ref, o_ref):
    o_ref[0, 0] = jnp.sum(p_ref[...]) * (1.0 / B)


@jax.jit
def _mean(partials):
    out = pl.pallas_call(
        _mean_body,
        out_shape=jax.ShapeDtypeStruct((1, 1), jnp.float32),
        out_specs=pl.BlockSpec(memory_space=pltpu.SMEM),
    )(partials)
    return out[0, 0]


def kernel(inputs, targets):
    flat = inputs.reshape(B * C)
    tgt = targets.astype(jnp.int32)
    partials = _sc_loss_partials(flat, tgt)
    return _mean(partials)


# speedup vs baseline: 1.4958x; 1.2519x over previous
import jax
import jax.numpy as jnp
from jax import lax
from jax.experimental import pallas as pl
from jax.experimental.pallas import tpu as pltpu
from jax.experimental.pallas import tpu_sc as plsc

B = 16384
C = 1000
NC = 2
NS = 16
NW = NC * NS
BPW = B // NW
R = 32
NCH = BPW // R
NBUF = 3

_IN_BOUNDS = "promise_in_bounds"
_LN2 = 0.6931471805599453
_SQRT2 = 1.4142135623730951


def _rot(x, lanes, sh):
    return x.at[(lanes + sh) & 15].get(mode=_IN_BOUNDS)


def _allsum(x, lanes):
    for sh in (8, 4, 2, 1):
        x = x + _rot(x, lanes, sh)
    return x


def _ln(r):
    bits = lax.bitcast_convert_type(r, jnp.int32)
    e = ((bits >> 23) & 0xFF) - 127
    m = lax.bitcast_convert_type((bits & 0x007FFFFF) | 0x3F800000,
                                 jnp.float32)
    big = m > _SQRT2
    m = jnp.where(big, m * 0.5, m)
    e = jnp.where(big, e + 1, e)
    f = m - 1.0
    s = f / (2.0 + f)
    s2 = s * s
    p = 2.0 * s * (1.0 + s2 * (1.0 / 3.0 + s2 * (0.2 + s2 * (1.0 / 7.0))))
    return e.astype(jnp.float32) * _LN2 + p


def _sc_body(x_hbm, tgt_hbm, out_hbm,
             tgt_v, b0, b1, b2, out_v, psum_v, s0, s1, s2, t0, t1, t2):
    bufs = [b0, b1, b2]
    sems = [s0, s1, s2]
    sems2 = [t0, t1, t2]
    wid = lax.axis_index("s") * NC + lax.axis_index("c")
    base = wid * BPW
    lanes = lax.iota(jnp.int32, 16)
    zeros16 = jnp.zeros((16,), jnp.int32)

    pltpu.sync_copy(tgt_hbm.at[pl.ds(base, BPW)], tgt_v)

    def start(ch):
        p = ch % NBUF
        slab0 = (base + ch * R) // 8
        h = R // 16
        return (pltpu.async_copy(x_hbm.at[pl.ds(slab0, h)],
                                 bufs[p].at[pl.ds(0, h)], sems[p]),
                pltpu.async_copy(x_hbm.at[pl.ds(slab0 + h, h)],
                                 bufs[p].at[pl.ds(h, h)], sems2[p]))

    handles = {}
    for ch in range(NBUF - 1):
        handles[ch] = start(ch)

    for ch in range(NCH):
        if ch + NBUF - 1 < NCH:
            handles[ch + NBUF - 1] = start(ch + NBUF - 1)
        handles[ch][0].wait()
        handles[ch][1].wait()
        buf = bufs[ch % NBUF]
        for gg in range(R // 16):
            g = ch * (R // 16) + gg
            sl = pl.ds(g * 16, 16)
            rloc = gg * 16 + lanes
            t16 = tgt_v[sl]
            xt = plsc.load_gather(buf, [rloc >> 3, rloc & 7, t16])
            x0 = plsc.load_gather(buf, [rloc >> 3, rloc & 7, zeros16])
            out16 = 1.0 + jnp.exp(x0 - xt)
            out_v[sl] = out16
            zmask = t16 == 0
            nz = _allsum(zmask.astype(jnp.int32), lanes)[0]

            @pl.when(nz > 0)
            def _():
                def colbody(c, acc):
                    col = plsc.load_gather(buf, [rloc >> 3, rloc & 7,
                                                 jnp.broadcast_to(c, (16,))])
                    return acc + jnp.exp(col - x0)
                acc = lax.fori_loop(0, C, colbody,
                                    jnp.zeros((16,), jnp.float32))
                out_v[sl] = jnp.where(zmask, acc, out16)

    acc_ln = jnp.zeros((16,), jnp.float32)
    for g in range(BPW // 16):
        acc_ln = acc_ln + _ln(out_v[pl.ds(g * 16, 16)])
    psum_v[...] = acc_ln
    pltpu.sync_copy(psum_v, out_hbm.at[wid])


@jax.jit
def _sc_loss_partials(x2d, tgt):
    mesh = plsc.VectorSubcoreMesh(core_axis_name="c", subcore_axis_name="s",
                                  num_cores=NC, num_subcores=NS)
    return pl.kernel(
        _sc_body,
        out_type=jax.ShapeDtypeStruct((NW, 16), jnp.float32),
        mesh=mesh,
        scratch_types=[
            pltpu.VMEM((BPW,), jnp.int32),
            pltpu.VMEM((R // 8, 8, C), jnp.float32),
            pltpu.VMEM((R // 8, 8, C), jnp.float32),
            pltpu.VMEM((R // 8, 8, C), jnp.float32),
            pltpu.VMEM((BPW,), jnp.float32),
            pltpu.VMEM((16,), jnp.float32),
            pltpu.SemaphoreType.DMA,
            pltpu.SemaphoreType.DMA,
            pltpu.SemaphoreType.DMA,
            pltpu.SemaphoreType.DMA,
            pltpu.SemaphoreType.DMA,
            pltpu.SemaphoreType.DMA,
        ],
        compiler_params=pltpu.CompilerParams(
            needs_layout_passes=False,
            use_tc_tiling_on_sc=True,
        ),
    )(x2d, tgt)


def kernel(inputs, targets):
    tgt = targets.astype(jnp.int32)
    partials = _sc_loss_partials(inputs.reshape(B // 8, 8, C), tgt)
    return jnp.sum(partials) * (1.0 / B)


# speedup vs baseline: 1.5328x; 1.0247x over previous
import jax
import jax.numpy as jnp
from jax import lax
from jax.experimental import pallas as pl
from jax.experimental.pallas import tpu as pltpu
from jax.experimental.pallas import tpu_sc as plsc

B = 16384
C = 1000
NC = 2
NS = 16
NW = NC * NS
BPW = B // NW
R = 16
NCH = BPW // R
NBUF = 6

_IN_BOUNDS = "promise_in_bounds"
_LN2 = 0.6931471805599453
_SQRT2 = 1.4142135623730951


def _rot(x, lanes, sh):
    return x.at[(lanes + sh) & 15].get(mode=_IN_BOUNDS)


def _allsum(x, lanes):
    for sh in (8, 4, 2, 1):
        x = x + _rot(x, lanes, sh)
    return x


def _ln(r):
    bits = lax.bitcast_convert_type(r, jnp.int32)
    e = ((bits >> 23) & 0xFF) - 127
    m = lax.bitcast_convert_type((bits & 0x007FFFFF) | 0x3F800000,
                                 jnp.float32)
    big = m > _SQRT2
    m = jnp.where(big, m * 0.5, m)
    e = jnp.where(big, e + 1, e)
    f = m - 1.0
    s = f / (2.0 + f)
    s2 = s * s
    p = 2.0 * s * (1.0 + s2 * (1.0 / 3.0 + s2 * (0.2 + s2 * (1.0 / 7.0))))
    return e.astype(jnp.float32) * _LN2 + p


def _sc_body(x_hbm, tgt_hbm, out_hbm,
             tgt_v, b0, b1, b2, b3, b4, b5, out_v, psum_v,
             s0, s1, s2, s3, s4, s5, t0, t1, t2, t3, t4, t5):
    bufs = [b0, b1, b2, b3, b4, b5]
    sems = [s0, s1, s2, s3, s4, s5]
    sems2 = [t0, t1, t2, t3, t4, t5]
    wid = lax.axis_index("s") * NC + lax.axis_index("c")
    base = wid * BPW
    lanes = lax.iota(jnp.int32, 16)
    zeros16 = jnp.zeros((16,), jnp.int32)

    pltpu.sync_copy(tgt_hbm.at[pl.ds(base, BPW)], tgt_v)

    def start(ch):
        p = ch % NBUF
        slab0 = (base + ch * R) // 8
        h = R // 16
        return (pltpu.async_copy(x_hbm.at[pl.ds(slab0, h)],
                                 bufs[p].at[pl.ds(0, h)], sems[p]),
                pltpu.async_copy(x_hbm.at[pl.ds(slab0 + h, h)],
                                 bufs[p].at[pl.ds(h, h)], sems2[p]))

    handles = {}
    for ch in range(NBUF - 1):
        handles[ch] = start(ch)

    for ch in range(NCH):
        if ch + NBUF - 1 < NCH:
            handles[ch + NBUF - 1] = start(ch + NBUF - 1)
        handles[ch][0].wait()
        handles[ch][1].wait()
        buf = bufs[ch % NBUF]
        for gg in range(R // 16):
            g = ch * (R // 16) + gg
            sl = pl.ds(g * 16, 16)
            rloc = gg * 16 + lanes
            t16 = tgt_v[sl]
            xt = plsc.load_gather(buf, [rloc >> 3, rloc & 7, t16])
            x0 = plsc.load_gather(buf, [rloc >> 3, rloc & 7, zeros16])
            out16 = 1.0 + jnp.exp(x0 - xt)
            out_v[sl] = out16
            zmask = t16 == 0
            nz = _allsum(zmask.astype(jnp.int32), lanes)[0]

            @pl.when(nz > 0)
            def _():
                def colbody(c, acc):
                    col = plsc.load_gather(buf, [rloc >> 3, rloc & 7,
                                                 jnp.broadcast_to(c, (16,))])
                    return acc + jnp.exp(col - x0)
                acc = lax.fori_loop(0, C, colbody,
                                    jnp.zeros((16,), jnp.float32))
                out_v[sl] = jnp.where(zmask, acc, out16)

    acc_ln = jnp.zeros((16,), jnp.float32)
    for g in range(BPW // 16):
        acc_ln = acc_ln + _ln(out_v[pl.ds(g * 16, 16)])
    psum_v[...] = acc_ln
    pltpu.sync_copy(psum_v, out_hbm.at[wid])


@jax.jit
def _sc_loss_partials(x2d, tgt):
    mesh = plsc.VectorSubcoreMesh(core_axis_name="c", subcore_axis_name="s",
                                  num_cores=NC, num_subcores=NS)
    return pl.kernel(
        _sc_body,
        out_type=jax.ShapeDtypeStruct((NW, 16), jnp.float32),
        mesh=mesh,
        scratch_types=[
            pltpu.VMEM((BPW,), jnp.int32),
            pltpu.VMEM((R // 8, 8, C), jnp.float32),
            pltpu.VMEM((R // 8, 8, C), jnp.float32),
            pltpu.VMEM((R // 8, 8, C), jnp.float32),
            pltpu.VMEM((R // 8, 8, C), jnp.float32),
            pltpu.VMEM((R // 8, 8, C), jnp.float32),
            pltpu.VMEM((R // 8, 8, C), jnp.float32),
            pltpu.VMEM((BPW,), jnp.float32),
            pltpu.VMEM((16,), jnp.float32),
            pltpu.SemaphoreType.DMA,
            pltpu.SemaphoreType.DMA,
            pltpu.SemaphoreType.DMA,
            pltpu.SemaphoreType.DMA,
            pltpu.SemaphoreType.DMA,
            pltpu.SemaphoreType.DMA,
            pltpu.SemaphoreType.DMA,
            pltpu.SemaphoreType.DMA,
            pltpu.SemaphoreType.DMA,
            pltpu.SemaphoreType.DMA,
            pltpu.SemaphoreType.DMA,
            pltpu.SemaphoreType.DMA,
        ],
        compiler_params=pltpu.CompilerParams(
            needs_layout_passes=False,
            use_tc_tiling_on_sc=True,
        ),
    )(x2d, tgt)


def kernel(inputs, targets):
    tgt = targets.astype(jnp.int32)
    partials = _sc_loss_partials(inputs.reshape(B // 8, 8, C), tgt)
    return jnp.sum(partials) * (1.0 / B)


# speedup vs baseline: 1.5440x; 1.0073x over previous
import jax
import jax.numpy as jnp
from jax import lax
from jax.experimental import pallas as pl
from jax.experimental.pallas import tpu as pltpu
from jax.experimental.pallas import tpu_sc as plsc

B = 16384
C = 1000
NC = 2
NS = 16
NW = NC * NS
BPW = B // NW
R = 16
NCH = BPW // R
NBUF = 7

_IN_BOUNDS = "promise_in_bounds"
_LN2 = 0.6931471805599453
_SQRT2 = 1.4142135623730951


def _rot(x, lanes, sh):
    return x.at[(lanes + sh) & 15].get(mode=_IN_BOUNDS)


def _allsum(x, lanes):
    for sh in (8, 4, 2, 1):
        x = x + _rot(x, lanes, sh)
    return x


def _ln(r):
    bits = lax.bitcast_convert_type(r, jnp.int32)
    e = ((bits >> 23) & 0xFF) - 127
    m = lax.bitcast_convert_type((bits & 0x007FFFFF) | 0x3F800000,
                                 jnp.float32)
    big = m > _SQRT2
    m = jnp.where(big, m * 0.5, m)
    e = jnp.where(big, e + 1, e)
    f = m - 1.0
    s = f / (2.0 + f)
    s2 = s * s
    p = 2.0 * s * (1.0 + s2 * (1.0 / 3.0 + s2 * (0.2 + s2 * (1.0 / 7.0))))
    return e.astype(jnp.float32) * _LN2 + p


def _sc_body(x_hbm, tgt_hbm, out_hbm,
             tgt_v, b0, b1, b2, b3, b4, b5, b6, out_v, psum_v,
             s0, s1, s2, s3, s4, s5, s6, t0, t1, t2, t3, t4, t5, t6):
    bufs = [b0, b1, b2, b3, b4, b5, b6]
    sems = [s0, s1, s2, s3, s4, s5, s6]
    sems2 = [t0, t1, t2, t3, t4, t5, t6]
    wid = lax.axis_index("s") * NC + lax.axis_index("c")
    base = wid * BPW
    lanes = lax.iota(jnp.int32, 16)
    zeros16 = jnp.zeros((16,), jnp.int32)

    pltpu.sync_copy(tgt_hbm.at[pl.ds(base, BPW)], tgt_v)

    def start(ch):
        p = ch % NBUF
        slab0 = (base + ch * R) // 8
        h = R // 16
        return (pltpu.async_copy(x_hbm.at[pl.ds(slab0, h)],
                                 bufs[p].at[pl.ds(0, h)], sems[p]),
                pltpu.async_copy(x_hbm.at[pl.ds(slab0 + h, h)],
                                 bufs[p].at[pl.ds(h, h)], sems2[p]))

    handles = {}
    for ch in range(NBUF - 1):
        handles[ch] = start(ch)

    for ch in range(NCH):
        if ch + NBUF - 1 < NCH:
            handles[ch + NBUF - 1] = start(ch + NBUF - 1)
        handles[ch][0].wait()
        handles[ch][1].wait()
        buf = bufs[ch % NBUF]
        for gg in range(R // 16):
            g = ch * (R // 16) + gg
            sl = pl.ds(g * 16, 16)
            rloc = gg * 16 + lanes
            t16 = tgt_v[sl]
            xt = plsc.load_gather(buf, [rloc >> 3, rloc & 7, t16])
            x0 = plsc.load_gather(buf, [rloc >> 3, rloc & 7, zeros16])
            out16 = 1.0 + jnp.exp(x0 - xt)
            out_v[sl] = out16
            zmask = t16 == 0
            nz = _allsum(zmask.astype(jnp.int32), lanes)[0]

            @pl.when(nz > 0)
            def _():
                def colbody(c, acc):
                    col = plsc.load_gather(buf, [rloc >> 3, rloc & 7,
                                                 jnp.broadcast_to(c, (16,))])
                    return acc + jnp.exp(col - x0)
                acc = lax.fori_loop(0, C, colbody,
                                    jnp.zeros((16,), jnp.float32))
                out_v[sl] = jnp.where(zmask, acc, out16)

    acc_ln = jnp.zeros((16,), jnp.float32)
    for g in range(BPW // 16):
        acc_ln = acc_ln + _ln(out_v[pl.ds(g * 16, 16)])
    psum_v[...] = acc_ln
    pltpu.sync_copy(psum_v, out_hbm.at[wid])


@jax.jit
def _sc_loss_partials(x2d, tgt):
    mesh = plsc.VectorSubcoreMesh(core_axis_name="c", subcore_axis_name="s",
                                  num_cores=NC, num_subcores=NS)
    return pl.kernel(
        _sc_body,
        out_type=jax.ShapeDtypeStruct((NW, 16), jnp.float32),
        mesh=mesh,
        scratch_types=[
            pltpu.VMEM((BPW,), jnp.int32),
            pltpu.VMEM((R // 8, 8, C), jnp.float32),
            pltpu.VMEM((R // 8, 8, C), jnp.float32),
            pltpu.VMEM((R // 8, 8, C), jnp.float32),
            pltpu.VMEM((R // 8, 8, C), jnp.float32),
            pltpu.VMEM((R // 8, 8, C), jnp.float32),
            pltpu.VMEM((R // 8, 8, C), jnp.float32),
            pltpu.VMEM((R // 8, 8, C), jnp.float32),
            pltpu.VMEM((BPW,), jnp.float32),
            pltpu.VMEM((16,), jnp.float32),
            pltpu.SemaphoreType.DMA,
            pltpu.SemaphoreType.DMA,
            pltpu.SemaphoreType.DMA,
            pltpu.SemaphoreType.DMA,
            pltpu.SemaphoreType.DMA,
            pltpu.SemaphoreType.DMA,
            pltpu.SemaphoreType.DMA,
            pltpu.SemaphoreType.DMA,
            pltpu.SemaphoreType.DMA,
            pltpu.SemaphoreType.DMA,
            pltpu.SemaphoreType.DMA,
            pltpu.SemaphoreType.DMA,
            pltpu.SemaphoreType.DMA,
            pltpu.SemaphoreType.DMA,
        ],
        compiler_params=pltpu.CompilerParams(
            needs_layout_passes=False,
            use_tc_tiling_on_sc=True,
        ),
    )(x2d, tgt)


def kernel(inputs, targets):
    tgt = targets.astype(jnp.int32)
    partials = _sc_loss_partials(inputs.reshape(B // 8, 8, C), tgt)
    return jnp.sum(partials) * (1.0 / B)
